# baseline (speedup 1.0000x reference)
import functools

import jax
import jax.numpy as jnp
from jax import lax
from jax.experimental import pallas as pl
from jax.experimental.pallas import tpu as pltpu
from jax.experimental.pallas import tpu_sc as plsc

_D = 64
_NC = 2
_NS = 16
_NW = _NC * _NS
_L = 16
_BT = 2
_GI = _BT * 128
_TW = _BT * 8 * 128
_NBUF = 3
_NBUFC = 6


def _make_convert(vocab: int):
    nvt_full = vocab // 128
    per_w_lo = nvt_full // _NW
    rem = nvt_full % _NW

    mesh = plsc.VectorSubcoreMesh(core_axis_name="c", subcore_axis_name="s")

    @functools.partial(
        pl.kernel,
        out_type=jax.ShapeDtypeStruct((vocab // 2, 128), jnp.float32),
        mesh=mesh,
        scratch_types=[
            pltpu.VMEM((_NBUFC, _D, 128), jnp.float32),
            pltpu.VMEM((_NBUFC, _D, 128), jnp.float32),
            [pltpu.SemaphoreType.DMA] * _NBUFC,
            [pltpu.SemaphoreType.DMA] * _NBUFC,
        ],
        compiler_params=pltpu.CompilerParams(
            use_tc_tiling_on_sc=True, needs_layout_passes=False
        ),
    )
    def convert(t3_hbm, tail_hbm, tab2_hbm, stage_v, w_v, sg, sw):
        wid = lax.axis_index("s") * _NC + lax.axis_index("c")
        nvt = per_w_lo + jnp.where(wid < rem, 1, 0)
        vt0 = wid * per_w_lo + jnp.minimum(wid, rem)
        iota = lax.iota(jnp.int32, _L)
        mvecs = [(iota + k) & (_L - 1) for k in range(_L)]
        svecs = [m * 128 + 2 * iota for m in mvecs]
        avecs = [iota * 128 + m for m in mvecs]

        def issue(g, b):
            pltpu.async_copy(
                t3_hbm.at[:, pl.ds((vt0 + g) * 128, 128)], stage_v.at[b], sg[b]
            )

        def wait_stage(b):
            pltpu.make_async_copy(
                t3_hbm.at[:, pl.ds(vt0 * 128, 128)], stage_v.at[b], sg[b]
            ).wait()

        def start_wb(g, b):
            pltpu.async_copy(
                w_v.at[b], tab2_hbm.at[pl.ds((vt0 + g) * _D, _D)], sw[b]
            )

        def wait_wb(b):
            pltpu.make_async_copy(
                w_v.at[b], tab2_hbm.at[pl.ds(vt0 * _D, _D)], sw[b]
            ).wait()

        def transpose(bdyn):
            sv = stage_v.at[bdyn]
            wv = w_v.at[bdyn]

            @plsc.parallel_loop(0, 8)
            def _(cq):
                c0 = cq * _L
                chi = c0 >> 6
                clo = c0 & 63
                srows = [mvecs[k] + clo for k in range(_L)]
                for i0 in range(0, _D, _L):
                    scol = 2 * iota + (2 * i0 + chi)
                    vs = [
                        plsc.load_gather(sv, [srows[k], scol])
                        for k in range(_L)
                    ]
                    arow = iota + i0
                    for k in range(_L):
                        plsc.store_scatter(
                            wv, [arow, mvecs[k] + c0], vs[k]
                        )

        for b in range(_NBUFC):
            issue(b, b)

        @pl.loop(0, per_w_lo + 1)
        def _(g):
            @pl.when(g < nvt)
            def _():
                bdyn = lax.rem(g, _NBUFC)

                for b in range(_NBUFC):
                    @pl.when(bdyn == b)
                    def _():
                        wait_stage(b)

                        @pl.when(g >= _NBUFC)
                        def _():
                            wait_wb(b)

                transpose(bdyn)

                for b in range(_NBUFC):
                    @pl.when(bdyn == b)
                    def _():
                        @pl.when(g + _NBUFC < nvt)
                        def _():
                            issue(g + _NBUFC, b)

                        start_wb(g, b)

        for b in range(_NBUFC):
            wait_wb(b)

        @pl.when(wid == _NW - 1)
        def _():
            pltpu.sync_copy(tail_hbm, stage_v.at[0, pl.ds(0, 32)])
            pltpu.sync_copy(
                stage_v.at[0, pl.ds(0, 32)],
                tab2_hbm.at[pl.ds(nvt_full * _D, 32)],
            )

    return convert


def _make_gather(h_dim: int, nbt: int):
    pair_total = h_dim * nbt // _BT
    per_w = pair_total // _NW
    pairs_per_h = nbt // _BT

    mesh = plsc.VectorSubcoreMesh(core_axis_name="c", subcore_axis_name="s")

    @functools.partial(
        pl.kernel,
        out_type=jax.ShapeDtypeStruct((h_dim, 8, nbt // _BT, _TW), jnp.float32),
        mesh=mesh,
        scratch_types=[
            pltpu.VMEM((_NBUF, _BT, 128), jnp.int32),
            pltpu.VMEM((_NBUF * _GI, _D), jnp.float32),
            pltpu.VMEM((_NBUF, 8 * _TW), jnp.float32),
            [pltpu.SemaphoreType.DMA] * _NBUF,
            [pltpu.SemaphoreType.DMA] * _NBUF,
        ],
        compiler_params=pltpu.CompilerParams(
            use_tc_tiling_on_sc=False, needs_layout_passes=False
        ),
    )
    def gather(xt_hbm, tab_hbm, out_hbm, idx_v, rows_v, t_v, sg, sw):
        wid = lax.axis_index("s") * _NC + lax.axis_index("c")
        gid0 = wid * per_w
        iota = lax.iota(jnp.int32, _L)
        mvecs = [(iota + k) & (_L - 1) for k in range(_L)]
        svecs = [(m >> 3) * _TW + (m & 7) * 128 + iota for m in mvecs]

        def coords(g):
            gid = gid0 + g
            h = gid // pairs_per_h
            btp = gid % pairs_per_h
            return h, btp

        def issue(g, b):
            h, btp = coords(g)
            pltpu.sync_copy(xt_hbm.at[h, pl.ds(btp * _BT, _BT)], idx_v.at[b])
            for j in range(_BT):
                pltpu.async_copy(
                    tab_hbm.at[idx_v.at[b, j]],
                    rows_v.at[pl.ds(b * _GI + j * 128, 128)],
                    sg[b],
                )

        def wait_gathers(b):
            for j in range(_BT):
                pltpu.make_async_copy(
                    tab_hbm.at[idx_v.at[b, j]],
                    rows_v.at[pl.ds(b * _GI + j * 128, 128)],
                    sg[b],
                ).wait()

        def transpose(bdyn):
            tv = t_v.at[bdyn]
            row0 = bdyn * _GI

            @plsc.parallel_loop(0, _D // _L)
            def _(dq):
                dd0 = dq * _L
                base_t = dq * (2 * _TW)
                colvs = [mvecs[k] + dd0 for k in range(_L)]
                for bb in range(_GI // _L):
                    btj = bb >> 3
                    bsg = bb & 7
                    rowv = iota + (row0 + bb * _L)
                    sbase = base_t + btj * 1024 + bsg * _L
                    vs = [
                        plsc.load_gather(rows_v, [rowv, colvs[k]])
                        for k in range(_L)
                    ]
                    for k in range(_L):
                        plsc.store_scatter(tv, [svecs[k] + sbase], vs[k])

        def start_wb(g, b):
            h, btp = coords(g)
            for dg in range(8):
                pltpu.async_copy(
                    t_v.at[b, pl.ds(dg * _TW, _TW)],
                    out_hbm.at[h, dg, btp],
                    sw[b],
                )

        def wait_wb(g, b):
            h, btp = coords(g)
            for dg in range(8):
                pltpu.make_async_copy(
                    t_v.at[b, pl.ds(dg * _TW, _TW)],
                    out_hbm.at[h, dg, btp],
                    sw[b],
                ).wait()

        for b in range(_NBUF):
            issue(b, b)

        @pl.loop(0, per_w)
        def _(g):
            bdyn = lax.rem(g, _NBUF)

            for b in range(_NBUF):
                @pl.when(bdyn == b)
                def _():
                    wait_gathers(b)

                    @pl.when(g >= _NBUF)
                    def _():
                        wait_wb(g, b)

            transpose(bdyn)

            for b in range(_NBUF):
                @pl.when(bdyn == b)
                def _():
                    @pl.when(g < per_w - _NBUF)
                    def _():
                        issue(g + _NBUF, b)

                    start_wb(g, b)

        for b in range(_NBUF):
            wait_wb(per_w - _NBUF + b, b)

    return gather


def kernel(x, table):
    batch, hist = x.shape
    vocab, d = table.shape
    nbt = batch // 128
    xt = jnp.transpose(x.astype(jnp.int32)).reshape(hist, nbt, 128)
    t3 = jnp.transpose(table)
    vfull = (vocab // 128) * 128
    tail2 = table[vfull:].reshape((vocab - vfull) // 2, 2 * d)
    tab2 = _make_convert(vocab)(t3, tail2)
    out4 = _make_gather(hist, nbt)(xt, tab2.reshape(vocab, d))
    out = (
        out4.reshape(hist, 8, nbt // _BT, _BT, 8, 128)
        .transpose((2, 3, 5, 0, 1, 4))
        .reshape(batch, hist, d)
    )
    return out


# speedup vs baseline: 1.1731x; 1.1731x over previous
import functools

import jax
import jax.numpy as jnp
from jax import lax
from jax.experimental import pallas as pl
from jax.experimental.pallas import tpu as pltpu
from jax.experimental.pallas import tpu_sc as plsc

_D = 64
_NC = 2
_NS = 16
_NW = _NC * _NS
_L = 16
_BT = 2
_GI = _BT * 128
_TW = _BT * 8 * 128
_NBUF = 3
_NBUFC = 4


def _make_convert(vocab: int):
    nvt_full = vocab // 128
    per_w_lo = nvt_full // _NW
    rem = nvt_full % _NW

    mesh = plsc.VectorSubcoreMesh(core_axis_name="c", subcore_axis_name="s")

    @functools.partial(
        pl.kernel,
        out_type=jax.ShapeDtypeStruct((vocab // 2, 128), jnp.float32),
        mesh=mesh,
        scratch_types=[
            pltpu.VMEM((_NBUFC, _D, 128), jnp.float32),
            pltpu.VMEM((_NBUFC, _D, 128), jnp.float32),
            [pltpu.SemaphoreType.DMA] * _NBUFC,
            [pltpu.SemaphoreType.DMA] * _NBUFC,
        ],
        compiler_params=pltpu.CompilerParams(
            use_tc_tiling_on_sc=True, needs_layout_passes=False
        ),
    )
    def convert(t3_hbm, tail_hbm, tab2_hbm, stage_v, w_v, sg, sw):
        wid = lax.axis_index("s") * _NC + lax.axis_index("c")
        nvt = per_w_lo + jnp.where(wid < rem, 1, 0)
        vt0 = wid * per_w_lo + jnp.minimum(wid, rem)
        iota = lax.iota(jnp.int32, _L)
        mvecs = [(iota + k) & (_L - 1) for k in range(_L)]
        svecs = [m * 128 + 2 * iota for m in mvecs]
        avecs = [iota * 128 + m for m in mvecs]

        def issue(g, b):
            pltpu.async_copy(
                t3_hbm.at[:, pl.ds((vt0 + g) * 128, 128)], stage_v.at[b], sg[b]
            )

        def wait_stage(b):
            pltpu.make_async_copy(
                t3_hbm.at[:, pl.ds(vt0 * 128, 128)], stage_v.at[b], sg[b]
            ).wait()

        def start_wb(g, b):
            pltpu.async_copy(
                w_v.at[b], tab2_hbm.at[pl.ds((vt0 + g) * _D, _D)], sw[b]
            )

        def wait_wb(b):
            pltpu.make_async_copy(
                w_v.at[b], tab2_hbm.at[pl.ds(vt0 * _D, _D)], sw[b]
            ).wait()

        def transpose(bdyn):
            sv = stage_v.at[bdyn]
            wv = w_v.at[bdyn]

            @plsc.parallel_loop(0, 8)
            def _(cq):
                c0 = cq * _L
                chi = c0 >> 6
                clo = c0 & 63
                srows = [mvecs[k] + clo for k in range(_L)]
                for i0 in range(0, _D, _L):
                    scol = 2 * iota + (2 * i0 + chi)
                    vs = [
                        plsc.load_gather(sv, [srows[k], scol])
                        for k in range(_L)
                    ]
                    arow = iota + i0
                    for k in range(_L):
                        plsc.store_scatter(
                            wv, [arow, mvecs[k] + c0], vs[k]
                        )

        for b in range(_NBUFC):
            issue(b, b)

        @pl.loop(0, per_w_lo + 1)
        def _(g):
            @pl.when(g < nvt)
            def _():
                bdyn = lax.rem(g, _NBUFC)

                for b in range(_NBUFC):
                    @pl.when(bdyn == b)
                    def _():
                        wait_stage(b)

                        @pl.when(g >= _NBUFC)
                        def _():
                            wait_wb(b)

                transpose(bdyn)

                for b in range(_NBUFC):
                    @pl.when(bdyn == b)
                    def _():
                        @pl.when(g + _NBUFC < nvt)
                        def _():
                            issue(g + _NBUFC, b)

                        start_wb(g, b)

        for b in range(_NBUFC):
            wait_wb(b)

        @pl.when(wid == _NW - 1)
        def _():
            pltpu.sync_copy(tail_hbm, stage_v.at[0, pl.ds(0, 32)])
            pltpu.sync_copy(
                stage_v.at[0, pl.ds(0, 32)],
                tab2_hbm.at[pl.ds(nvt_full * _D, 32)],
            )

    return convert


def _make_gather(h_dim: int, nbt: int):
    pair_total = h_dim * nbt // _BT
    per_w = pair_total // _NW
    pairs_per_h = nbt // _BT

    mesh = plsc.VectorSubcoreMesh(core_axis_name="c", subcore_axis_name="s")

    @functools.partial(
        pl.kernel,
        out_type=jax.ShapeDtypeStruct((h_dim, 8, nbt // _BT, _TW), jnp.float32),
        mesh=mesh,
        scratch_types=[
            pltpu.VMEM((_NBUF, _BT, 128), jnp.int32),
            pltpu.VMEM((_NBUF * _GI, _D), jnp.float32),
            pltpu.VMEM((_NBUF, 8 * _TW), jnp.float32),
            [pltpu.SemaphoreType.DMA] * _NBUF,
            [pltpu.SemaphoreType.DMA] * _NBUF,
        ],
        compiler_params=pltpu.CompilerParams(
            use_tc_tiling_on_sc=False, needs_layout_passes=False
        ),
    )
    def gather(xt_hbm, tab_hbm, out_hbm, idx_v, rows_v, t_v, sg, sw):
        wid = lax.axis_index("s") * _NC + lax.axis_index("c")
        gid0 = wid * per_w
        iota = lax.iota(jnp.int32, _L)
        mvecs = [(iota + k) & (_L - 1) for k in range(_L)]
        svecs = [(m >> 3) * _TW + (m & 7) * 128 + iota for m in mvecs]

        def coords(g):
            gid = gid0 + g
            h = gid // pairs_per_h
            btp = gid % pairs_per_h
            return h, btp

        def issue(g, b):
            h, btp = coords(g)
            pltpu.sync_copy(xt_hbm.at[h, pl.ds(btp * _BT, _BT)], idx_v.at[b])
            for j in range(_BT):
                pltpu.async_copy(
                    tab_hbm.at[idx_v.at[b, j]],
                    rows_v.at[pl.ds(b * _GI + j * 128, 128)],
                    sg[b],
                )

        def wait_gathers(b):
            for j in range(_BT):
                pltpu.make_async_copy(
                    tab_hbm.at[idx_v.at[b, j]],
                    rows_v.at[pl.ds(b * _GI + j * 128, 128)],
                    sg[b],
                ).wait()

        def transpose(bdyn):
            tv = t_v.at[bdyn]
            row0 = bdyn * _GI

            @plsc.parallel_loop(0, _D // _L)
            def _(dq):
                dd0 = dq * _L
                base_t = dq * (2 * _TW)
                colvs = [mvecs[k] + dd0 for k in range(_L)]
                for bb in range(_GI // _L):
                    btj = bb >> 3
                    bsg = bb & 7
                    rowv = iota + (row0 + bb * _L)
                    sbase = base_t + btj * 1024 + bsg * _L
                    vs = [
                        plsc.load_gather(rows_v, [rowv, colvs[k]])
                        for k in range(_L)
                    ]
                    for k in range(_L):
                        plsc.store_scatter(tv, [svecs[k] + sbase], vs[k])

        def start_wb(g, b):
            h, btp = coords(g)
            for dg in range(8):
                pltpu.async_copy(
                    t_v.at[b, pl.ds(dg * _TW, _TW)],
                    out_hbm.at[h, dg, btp],
                    sw[b],
                )

        def wait_wb(g, b):
            h, btp = coords(g)
            for dg in range(8):
                pltpu.make_async_copy(
                    t_v.at[b, pl.ds(dg * _TW, _TW)],
                    out_hbm.at[h, dg, btp],
                    sw[b],
                ).wait()

        for b in range(_NBUF):
            issue(b, b)

        @pl.loop(0, per_w)
        def _(g):
            bdyn = lax.rem(g, _NBUF)

            for b in range(_NBUF):
                @pl.when(bdyn == b)
                def _():
                    wait_gathers(b)

                    @pl.when(g >= _NBUF)
                    def _():
                        wait_wb(g, b)

            transpose(bdyn)

            for b in range(_NBUF):
                @pl.when(bdyn == b)
                def _():
                    @pl.when(g < per_w - _NBUF)
                    def _():
                        issue(g + _NBUF, b)

                    start_wb(g, b)

        for b in range(_NBUF):
            wait_wb(per_w - _NBUF + b, b)

    return gather


def kernel(x, table):
    batch, hist = x.shape
    vocab, d = table.shape
    nbt = batch // 128
    xt = jnp.transpose(x.astype(jnp.int32)).reshape(hist, nbt, 128)
    t3 = jnp.transpose(table)
    vfull = (vocab // 128) * 128
    tail2 = table[vfull:].reshape((vocab - vfull) // 2, 2 * d)
    tab2 = _make_convert(vocab)(t3, tail2)
    out4 = _make_gather(hist, nbt)(xt, tab2.reshape(vocab, d))
    out = (
        out4.reshape(hist, 8, nbt // _BT, _BT, 8, 128)
        .transpose((2, 3, 5, 0, 1, 4))
        .reshape(batch, hist, d)
    )
    return out
